# baseline (speedup 1.0000x reference)
import functools

import jax
import jax.numpy as jnp
from jax import lax
from jax.experimental import pallas as pl
from jax.experimental.pallas import tpu as pltpu
from jax.experimental.pallas import tpu_sc as plsc

N_ROOTS = 20
N_CHILD = 9
N_CLASSES = 200
B = 16
S = 8192
PRED_THRESH = 0.5

T = 128
NGRP = T // 16
SPAN = 256


def _tree_predict_chunk(in_v, out_v, idx_v):
    lane = lax.iota(jnp.int32, 16)
    zero_f = jnp.zeros((16,), jnp.float32)
    zero_i = jnp.zeros((16,), jnp.int32)
    for j in range(NGRP):
        col = lane + (j * 16)
        roots = [in_v[k, pl.ds(j * 16, 16)] for k in range(N_ROOTS)]
        m = roots[0]
        ridx = zero_i
        for k in range(1, N_ROOTS):
            gt = roots[k] > m
            m = jnp.where(gt, roots[k], m)
            ridx = jnp.where(gt, jnp.full((16,), k, jnp.int32), ridx)
        s = jnp.exp(roots[0] - m)
        for k in range(1, N_ROOTS):
            s = s + jnp.exp(roots[k] - m)
        h_r = 1.0 / s
        rowbase = ridx * N_CHILD + N_ROOTS
        ch = [plsc.load_gather(in_v, [rowbase + jj, col])
              for jj in range(N_CHILD)]
        cm = ch[0]
        cidx = zero_i
        for jj in range(1, N_CHILD):
            gt = ch[jj] > cm
            cm = jnp.where(gt, ch[jj], cm)
            cidx = jnp.where(gt, jnp.full((16,), jj, jnp.int32), cidx)
        cs = jnp.exp(ch[0] - cm)
        for jj in range(1, N_CHILD):
            cs = cs + jnp.exp(ch[jj] - cm)
        h_c = (1.0 / cs) * h_r
        rmask = h_r > PRED_THRESH
        vr = jnp.where(rmask, h_r, zero_f)
        vc = jnp.where(rmask & (h_c > PRED_THRESH), h_c, zero_f)
        crow = rowbase + cidx
        plsc.store_scatter(out_v, [ridx, col], vr)
        plsc.store_scatter(out_v, [crow, col], vc)
        idx_v[2 * j, :] = ridx
        idx_v[2 * j + 1, :] = crow


def _clean_chunk(out_v, idx_v):
    lane = lax.iota(jnp.int32, 16)
    zero_f = jnp.zeros((16,), jnp.float32)
    for j in range(NGRP):
        col = lane + (j * 16)
        plsc.store_scatter(out_v, [idx_v[2 * j, :], col], zero_f)
        plsc.store_scatter(out_v, [idx_v[2 * j + 1, :], col], zero_f)


def _sc_body(x_hbm, out_hbm, in0, in1, o0, o1, idx0, idx1,
             si0, si1, so0, so1):
    info = plsc.get_sparse_core_info()
    nc = info.num_cores
    wid = lax.axis_index("s") * nc + lax.axis_index("c")
    base_s = wid * SPAN
    ins = (in0, in1)
    outs = (o0, o1)
    idxs = (idx0, idx1)
    sis = (si0, si1)
    sos = (so0, so1)

    zero_f = jnp.zeros((16,), jnp.float32)
    for ov in outs:
        for k in range(N_CLASSES):
            for j in range(NGRP):
                ov[k, pl.ds(j * 16, 16)] = zero_f

    for p in range(2):
        pltpu.async_copy(
            x_hbm.at[0, :, pl.ds(base_s + p * T, T)], ins[p], sis[p])

    def u_body(u, carry):
        for p in range(2):
            off = base_s + p * T
            pltpu.make_async_copy(
                x_hbm.at[u, :, pl.ds(off, T)], ins[p], sis[p]).wait()

            @pl.when(u > 0)
            def _wait_clean():
                pltpu.make_async_copy(
                    outs[p], out_hbm.at[u - 1, :, pl.ds(off, T)],
                    sos[p]).wait()
                _clean_chunk(outs[p], idxs[p])

            _tree_predict_chunk(ins[p], outs[p], idxs[p])
            pltpu.async_copy(outs[p], out_hbm.at[u, :, pl.ds(off, T)],
                             sos[p])

            @pl.when(u < B - 1)
            def _prefetch():
                pltpu.async_copy(
                    x_hbm.at[u + 1, :, pl.ds(off, T)], ins[p], sis[p])
        return carry

    lax.fori_loop(0, B, u_body, 0)

    for p in range(2):
        pltpu.make_async_copy(
            outs[p], out_hbm.at[B - 1, :, pl.ds(base_s + p * T, T)],
            sos[p]).wait()


@jax.jit
def _sc_tree_predict(x):
    mesh = plsc.VectorSubcoreMesh(core_axis_name="c", subcore_axis_name="s")
    k = functools.partial(
        pl.kernel,
        mesh=mesh,
        compiler_params=pltpu.CompilerParams(needs_layout_passes=False),
        out_type=jax.ShapeDtypeStruct((B, N_CLASSES, S), jnp.float32),
        scratch_types=[
            pltpu.VMEM((N_CLASSES, T), jnp.float32),
            pltpu.VMEM((N_CLASSES, T), jnp.float32),
            pltpu.VMEM((N_CLASSES, T), jnp.float32),
            pltpu.VMEM((N_CLASSES, T), jnp.float32),
            pltpu.VMEM((2 * NGRP, 16), jnp.int32),
            pltpu.VMEM((2 * NGRP, 16), jnp.int32),
            pltpu.SemaphoreType.DMA,
            pltpu.SemaphoreType.DMA,
            pltpu.SemaphoreType.DMA,
            pltpu.SemaphoreType.DMA,
        ],
    )(_sc_body)
    return k(x)


def _tc_copy_body(x_ref, o_ref):
    o_ref[...] = x_ref[...] * 2.0


@jax.jit
def _tc_zero_probe(x):
    return pl.pallas_call(
        _tc_copy_body,
        out_shape=jax.ShapeDtypeStruct((B, N_CLASSES, S // 2), jnp.float32),
        grid=(B, 8),
        in_specs=[pl.BlockSpec(
            (1, N_CLASSES, S // 16), lambda i, j: (i, 0, j))],
        out_specs=pl.BlockSpec(
            (1, N_CLASSES, S // 16), lambda i, j: (i, 0, j)),
    )(x[:, :, : S // 2])


def kernel(x):
    return (_sc_tree_predict(x), _tc_zero_probe(x))


# speedup vs baseline: 1.5796x; 1.5796x over previous
import functools

import jax
import jax.numpy as jnp
from jax import lax
from jax.experimental import pallas as pl
from jax.experimental.pallas import tpu as pltpu
from jax.experimental.pallas import tpu_sc as plsc

N_ROOTS = 20
N_CHILD = 9
N_CLASSES = 200
B = 16
S = 8192
PRED_THRESH = 0.5

T = 128
NGRP = T // 16
SPAN = 256


def _tree_predict_chunk(in_v, out_v, idx_v):
    lane = lax.iota(jnp.int32, 16)
    zero_f = jnp.zeros((16,), jnp.float32)
    zero_i = jnp.zeros((16,), jnp.int32)
    for j in range(NGRP):
        col = lane + (j * 16)
        roots = [in_v[k, pl.ds(j * 16, 16)] for k in range(N_ROOTS)]
        m = roots[0]
        ridx = zero_i
        for k in range(1, N_ROOTS):
            gt = roots[k] > m
            m = jnp.where(gt, roots[k], m)
            ridx = jnp.where(gt, jnp.full((16,), k, jnp.int32), ridx)
        s = jnp.exp(roots[0] - m)
        for k in range(1, N_ROOTS):
            s = s + jnp.exp(roots[k] - m)
        h_r = 1.0 / s
        rowbase = ridx * N_CHILD + N_ROOTS
        ch = [plsc.load_gather(in_v, [rowbase + jj, col])
              for jj in range(N_CHILD)]
        cm = ch[0]
        cidx = zero_i
        for jj in range(1, N_CHILD):
            gt = ch[jj] > cm
            cm = jnp.where(gt, ch[jj], cm)
            cidx = jnp.where(gt, jnp.full((16,), jj, jnp.int32), cidx)
        cs = jnp.exp(ch[0] - cm)
        for jj in range(1, N_CHILD):
            cs = cs + jnp.exp(ch[jj] - cm)
        h_c = (1.0 / cs) * h_r
        rmask = h_r > PRED_THRESH
        vr = jnp.where(rmask, h_r, zero_f)
        vc = jnp.where(rmask & (h_c > PRED_THRESH), h_c, zero_f)
        crow = rowbase + cidx
        plsc.store_scatter(out_v, [ridx, col], vr)
        plsc.store_scatter(out_v, [crow, col], vc)
        idx_v[2 * j, :] = ridx
        idx_v[2 * j + 1, :] = crow


def _clean_chunk(out_v, idx_v):
    lane = lax.iota(jnp.int32, 16)
    zero_f = jnp.zeros((16,), jnp.float32)
    for j in range(NGRP):
        col = lane + (j * 16)
        plsc.store_scatter(out_v, [idx_v[2 * j, :], col], zero_f)
        plsc.store_scatter(out_v, [idx_v[2 * j + 1, :], col], zero_f)


def _sc_body(x_hbm, out_hbm, in0, in1, o0, o1, idx0, idx1,
             si0, si1, so0, so1):
    info = plsc.get_sparse_core_info()
    nc = info.num_cores
    wid = lax.axis_index("s") * nc + lax.axis_index("c")
    base_s = wid * SPAN
    ins = (in0, in1)
    outs = (o0, o1)
    idxs = (idx0, idx1)
    sis = (si0, si1)
    sos = (so0, so1)

    zero_f = jnp.zeros((16,), jnp.float32)
    for ov in outs:
        for k in range(N_CLASSES):
            for j in range(NGRP):
                ov[k, pl.ds(j * 16, 16)] = zero_f

    for p in range(2):
        pltpu.async_copy(
            x_hbm.at[0, :, pl.ds(base_s + p * T, T)], ins[p], sis[p])

    def u_body(u, carry):
        for p in range(2):
            off = base_s + p * T
            pltpu.make_async_copy(
                x_hbm.at[u, :, pl.ds(off, T)], ins[p], sis[p]).wait()

            @pl.when(u > 0)
            def _wait_clean():
                pltpu.make_async_copy(
                    outs[p], out_hbm.at[u - 1, :, pl.ds(off, T)],
                    sos[p]).wait()
                _clean_chunk(outs[p], idxs[p])

            _tree_predict_chunk(ins[p], outs[p], idxs[p])
            pltpu.async_copy(outs[p], out_hbm.at[u, :, pl.ds(off, T)],
                             sos[p])

            @pl.when(u < B - 1)
            def _prefetch():
                pltpu.async_copy(
                    x_hbm.at[u + 1, :, pl.ds(off, T)], ins[p], sis[p])
        return carry

    lax.fori_loop(0, B, u_body, 0)

    for p in range(2):
        pltpu.make_async_copy(
            outs[p], out_hbm.at[B - 1, :, pl.ds(base_s + p * T, T)],
            sos[p]).wait()


@jax.jit
def _sc_tree_predict(x):
    mesh = plsc.VectorSubcoreMesh(core_axis_name="c", subcore_axis_name="s")
    k = functools.partial(
        pl.kernel,
        mesh=mesh,
        compiler_params=pltpu.CompilerParams(needs_layout_passes=False),
        out_type=jax.ShapeDtypeStruct((B, N_CLASSES, S), jnp.float32),
        scratch_types=[
            pltpu.VMEM((N_CLASSES, T), jnp.float32),
            pltpu.VMEM((N_CLASSES, T), jnp.float32),
            pltpu.VMEM((N_CLASSES, T), jnp.float32),
            pltpu.VMEM((N_CLASSES, T), jnp.float32),
            pltpu.VMEM((2 * NGRP, 16), jnp.int32),
            pltpu.VMEM((2 * NGRP, 16), jnp.int32),
            pltpu.SemaphoreType.DMA,
            pltpu.SemaphoreType.DMA,
            pltpu.SemaphoreType.DMA,
            pltpu.SemaphoreType.DMA,
        ],
    )(_sc_body)
    return k(x)


def kernel(x):
    return _sc_tree_predict(x)
